# baseline (speedup 1.0000x reference)
import functools

import jax
import jax.numpy as jnp
from jax import lax
from jax.experimental import pallas as pl
from jax.experimental.pallas import tpu as pltpu
from jax.experimental.pallas import tpu_sc as plsc

_SC_CORES = 2
_SC_SUBCORES = 16
_SC_WORKERS = _SC_CORES * _SC_SUBCORES
_SC_CHUNK = 32

_TM = 128


def _router_pos(x, w_router):
    t, d = x.shape
    e = w_router.shape[1]
    nt = t // _TM

    def body(x_ref, wr_ref, gate16_ref, pos_ref, go_ref, eid_s, r1_s):
        logits = jnp.dot(x_ref[...], wr_ref[...],
                         preferred_element_type=jnp.float32)
        m = jnp.max(logits, axis=-1, keepdims=True)
        ssum = jnp.sum(jnp.exp(logits - m), axis=-1, keepdims=True)
        gate16_ref[...] = jnp.broadcast_to(1.0 / ssum, (t, 128))
        eid_s[...] = jnp.argmax(logits, axis=-1, keepdims=True).astype(
            jnp.int32)
        ri = lax.broadcasted_iota(jnp.int32, (_TM, _TM), 0)
        ci = lax.broadcasted_iota(jnp.int32, (_TM, _TM), 1)
        tri = (ci < ri).astype(jnp.float32)
        cols = lax.broadcasted_iota(jnp.int32, (_TM, e), 1)
        carry = jnp.zeros((1, e), jnp.float32)
        for tt in range(nt):
            eid_t = eid_s[pl.ds(tt * _TM, _TM), :]
            onehot = (cols == eid_t).astype(jnp.float32)
            cum = jnp.dot(tri, onehot, preferred_element_type=jnp.float32)
            r1_t = jnp.sum((cum + carry) * onehot, axis=-1, keepdims=True)
            r1_s[pl.ds(tt * _TM, _TM), :] = r1_t
            carry = carry + jnp.sum(onehot, axis=0, keepdims=True)
        gri = lax.broadcasted_iota(jnp.int32, (e, e), 0)
        gci = lax.broadcasted_iota(jnp.int32, (e, e), 1)
        tri_e = (gri < gci).astype(jnp.float32)
        go_f = jnp.dot(carry, tri_e,
                       preferred_element_type=jnp.float32)
        go_ref[...] = go_f.astype(jnp.int32)
        for tt in range(nt):
            eid_t = eid_s[pl.ds(tt * _TM, _TM), :]
            onehot = (cols == eid_t).astype(jnp.float32)
            base = jnp.sum(onehot * go_f, axis=-1, keepdims=True)
            pos_ref[pl.ds(tt * _TM, _TM), :] = (
                base + r1_s[pl.ds(tt * _TM, _TM), :]).astype(jnp.int32)

    return pl.pallas_call(
        body,
        out_shape=(jax.ShapeDtypeStruct((t, 128), jnp.float32),
                   jax.ShapeDtypeStruct((t, 1), jnp.int32),
                   jax.ShapeDtypeStruct((1, e), jnp.int32)),
        scratch_shapes=[pltpu.VMEM((t, 1), jnp.int32),
                        pltpu.VMEM((t, 1), jnp.float32)],
    )(x, w_router)


def _slot_metadata(go_e, t, num_experts, tm, num_slots):
    go = jnp.concatenate([go_e, jnp.full((1,), t, jnp.int32)])
    counts = go[1:] - go[:num_experts]
    first_tile = go[:num_experts] // tm
    last_tile = (jnp.maximum(go[1:], 1) - 1) // tm
    span = jnp.where(counts > 0, last_tile - first_tile + 1, 0).astype(jnp.int32)
    cum = jnp.cumsum(span, dtype=jnp.int32)
    slot_start = cum - span
    total = cum[num_experts - 1]

    i = jnp.arange(num_slots, dtype=jnp.int32)
    g_i = jnp.searchsorted(cum, i, side="right").astype(jnp.int32)
    g_i = jnp.minimum(g_i, num_experts - 1)
    tile_i = first_tile[g_i] + (i - slot_start[g_i])
    valid = i < total
    g_last = jnp.searchsorted(cum, total - 1, side="right").astype(jnp.int32)
    g_last = jnp.minimum(g_last, num_experts - 1)
    tile_last = last_tile[g_last]
    g_meta = jnp.where(valid, g_i, g_last).astype(jnp.int32)
    t_meta = jnp.where(valid, tile_i, tile_last).astype(jnp.int32)
    lo = jnp.where(valid, go[g_meta], 0).astype(jnp.int32)
    hi = jnp.where(valid, go[jnp.minimum(g_meta + 1, num_experts)], 0)
    hi = jnp.where(valid, hi, 0).astype(jnp.int32)
    prev_tile = jnp.concatenate([jnp.full((1,), -1, jnp.int32), t_meta[:-1]])
    first = (valid & (t_meta != prev_tile)).astype(jnp.int32)
    return g_meta, t_meta, lo, hi, first


def _sc_dispatch(x, pos3, gate16):
    t, d = x.shape
    rows_per_w = t // _SC_WORKERS
    n_ch = rows_per_w // _SC_CHUNK
    mesh = plsc.VectorSubcoreMesh(core_axis_name="c", subcore_axis_name="s")

    @functools.partial(
        pl.kernel, mesh=mesh,
        out_type=(jax.ShapeDtypeStruct((t, d), jnp.float32),
                  jax.ShapeDtypeStruct((t, 128), jnp.float32)),
        scratch_types=[
            pltpu.VMEM((n_ch, _SC_CHUNK), jnp.int32),
            pltpu.VMEM((_SC_CHUNK, d), jnp.float32),
            pltpu.VMEM((_SC_CHUNK, 128), jnp.float32),
            pltpu.SemaphoreType.DMA,
        ],
    )
    def k(x_hbm, pos3_hbm, g16_hbm, xs_hbm, g16s_hbm,
          pos2_v, rows_v, grows_v, sem):
        wid = lax.axis_index("s") * _SC_CORES + lax.axis_index("c")
        base = wid * rows_per_w
        pltpu.sync_copy(pos3_hbm.at[wid], pos2_v)
        for c in range(n_ch):
            pltpu.sync_copy(x_hbm.at[pl.ds(base + c * _SC_CHUNK, _SC_CHUNK)],
                            rows_v)
            pltpu.sync_copy(g16_hbm.at[pl.ds(base + c * _SC_CHUNK, _SC_CHUNK)],
                            grows_v)
            cp1 = pltpu.async_copy(rows_v, xs_hbm.at[pos2_v.at[c]], sem)
            cp2 = pltpu.async_copy(grows_v, g16s_hbm.at[pos2_v.at[c]], sem)
            cp1.wait()
            cp2.wait()

    return k(x, pos3, gate16)


def _sc_gather(x, idx):
    t, d = x.shape
    rows_per_w = t // _SC_WORKERS
    n_ch = rows_per_w // _SC_CHUNK
    mesh = plsc.VectorSubcoreMesh(core_axis_name="c", subcore_axis_name="s")

    @functools.partial(
        pl.kernel, mesh=mesh,
        out_type=jax.ShapeDtypeStruct((t, d), jnp.float32),
        scratch_types=[
            pltpu.VMEM((rows_per_w,), jnp.int32),
            pltpu.VMEM((_SC_CHUNK, d), jnp.float32),
            pltpu.SemaphoreType.DMA,
        ],
    )
    def k(x_hbm, idx_hbm, out_hbm, idx_v, rows_v, sem):
        wid = lax.axis_index("s") * _SC_CORES + lax.axis_index("c")
        base = wid * rows_per_w
        pltpu.sync_copy(idx_hbm.at[pl.ds(base, rows_per_w)], idx_v)
        for c in range(n_ch):
            pltpu.async_copy(
                x_hbm.at[idx_v.at[pl.ds(c * _SC_CHUNK, _SC_CHUNK)]],
                rows_v, sem).wait()
            pltpu.sync_copy(rows_v, out_hbm.at[pl.ds(base + c * _SC_CHUNK,
                                                     _SC_CHUNK)])

    return k(x, idx)


def _grouped_mlp(xs, gate2, w1, b1r, w2, b2r, g_meta, t_meta, lo, hi, first):
    t, d = xs.shape
    e, _, f = w1.shape
    num_slots = g_meta.shape[0]

    nf = 2
    fb = f // nf

    def body(g_ref, t_ref, lo_ref, hi_ref, first_ref,
             xs_ref, w1_ref, b1_ref, w2_ref, b2_ref, gate_ref, out_ref):
        i = pl.program_id(0)
        j = pl.program_id(1)
        row0 = t_ref[i] * _TM
        ridx = row0 + lax.broadcasted_iota(jnp.int32, (_TM, 1), 0)
        mask = (ridx >= lo_ref[i]) & (ridx < hi_ref[i])
        h = jnp.dot(xs_ref[...], w1_ref[0],
                    preferred_element_type=jnp.float32) + b1_ref[0]
        h = jax.nn.gelu(h)
        y = jnp.dot(h, w2_ref[0], preferred_element_type=jnp.float32)
        y = jnp.where(j == 0, y + b2_ref[0], y)
        scale = jnp.where(mask, gate_ref[:, :1], 0.0)
        contrib = y * scale

        @pl.when((first_ref[i] != 0) & (j == 0))
        def _init():
            out_ref[...] = contrib

        @pl.when((first_ref[i] == 0) | (j != 0))
        def _accum():
            out_ref[...] += contrib

    grid_spec = pltpu.PrefetchScalarGridSpec(
        num_scalar_prefetch=5,
        grid=(num_slots, nf),
        in_specs=[
            pl.BlockSpec((_TM, d),
                         lambda i, j, g, tt, lo_, hi_, fr: (tt[i], 0)),
            pl.BlockSpec((1, d, fb),
                         lambda i, j, g, tt, lo_, hi_, fr: (g[i], 0, j)),
            pl.BlockSpec((1, 1, fb),
                         lambda i, j, g, tt, lo_, hi_, fr: (g[i], 0, j)),
            pl.BlockSpec((1, fb, d),
                         lambda i, j, g, tt, lo_, hi_, fr: (g[i], j, 0)),
            pl.BlockSpec((1, 1, d),
                         lambda i, j, g, tt, lo_, hi_, fr: (g[i], 0, 0)),
            pl.BlockSpec((_TM, 128),
                         lambda i, j, g, tt, lo_, hi_, fr: (tt[i], 0)),
        ],
        out_specs=pl.BlockSpec((_TM, d),
                               lambda i, j, g, tt, lo_, hi_, fr: (tt[i], 0)),
    )
    return pl.pallas_call(
        body,
        grid_spec=grid_spec,
        out_shape=jax.ShapeDtypeStruct((t, d), jnp.float32),
        compiler_params=pltpu.CompilerParams(
            dimension_semantics=("arbitrary", "arbitrary")),
    )(g_meta, t_meta, lo, hi, first, xs, w1, b1r, w2, b2r, gate2)


def kernel(hidden_states, W_router, W1, b1, W2, b2):
    s, b, d = hidden_states.shape
    e, _, f = W1.shape
    x = hidden_states.reshape(-1, d)
    t = x.shape[0]
    nt = t // _TM
    num_slots = nt + e

    gate16, pos, go = _router_pos(x, W_router)
    go_e = go.reshape(-1)

    g_meta, t_meta, lo, hi, first = _slot_metadata(
        go_e, t, e, _TM, num_slots)

    pos_flat = pos.reshape(-1)
    rows_per_w = t // _SC_WORKERS
    n_ch = rows_per_w // _SC_CHUNK
    pos3 = pos_flat.reshape(_SC_WORKERS, n_ch, _SC_CHUNK)
    xs, g16s = _sc_dispatch(x, pos3, gate16)

    b1r = b1.reshape(e, 1, f)
    b2r = b2.reshape(e, 1, d)
    ys = _grouped_mlp(xs, g16s, W1, b1r, W2, b2r,
                      g_meta, t_meta, lo, hi, first)

    out = _sc_gather(ys, pos_flat)
    return out.reshape(s, b, d)


# speedup vs baseline: 1.1717x; 1.1717x over previous
import functools

import jax
import jax.numpy as jnp
from jax import lax
from jax.experimental import pallas as pl
from jax.experimental.pallas import tpu as pltpu
from jax.experimental.pallas import tpu_sc as plsc

_SC_CORES = 2
_SC_SUBCORES = 16
_SC_WORKERS = _SC_CORES * _SC_SUBCORES
_SC_CHUNK = 32

_TM = 128


def _router_pos(x, w_router):
    t, d = x.shape
    e = w_router.shape[1]
    nt = t // _TM

    def body(x_ref, wr_ref, gate16_ref, pos_ref, gmeta_ref, tmeta_ref,
             lo_ref, hi_ref, first_ref, eid_s, r1_s):
        logits = jnp.dot(x_ref[...], wr_ref[...],
                         preferred_element_type=jnp.float32)
        m = jnp.max(logits, axis=-1, keepdims=True)
        ssum = jnp.sum(jnp.exp(logits - m), axis=-1, keepdims=True)
        gate16_ref[...] = jnp.broadcast_to(1.0 / ssum, (t, 128))
        eid_s[...] = jnp.argmax(logits, axis=-1, keepdims=True).astype(
            jnp.int32)
        ri = lax.broadcasted_iota(jnp.int32, (_TM, _TM), 0)
        ci = lax.broadcasted_iota(jnp.int32, (_TM, _TM), 1)
        tri = (ci < ri).astype(jnp.float32)
        cols = lax.broadcasted_iota(jnp.int32, (_TM, e), 1)
        carry = jnp.zeros((1, e), jnp.float32)
        for tt in range(nt):
            eid_t = eid_s[pl.ds(tt * _TM, _TM), :]
            onehot = (cols == eid_t).astype(jnp.float32)
            cum = jnp.dot(tri, onehot, preferred_element_type=jnp.float32)
            r1_t = jnp.sum((cum + carry) * onehot, axis=-1, keepdims=True)
            r1_s[pl.ds(tt * _TM, _TM), :] = r1_t
            carry = carry + jnp.sum(onehot, axis=0, keepdims=True)
        gri = lax.broadcasted_iota(jnp.int32, (e, e), 0)
        gci = lax.broadcasted_iota(jnp.int32, (e, e), 1)
        tri_e = (gri < gci).astype(jnp.float32)
        go_f = jnp.dot(carry, tri_e,
                       preferred_element_type=jnp.float32)
        for tt in range(nt):
            eid_t = eid_s[pl.ds(tt * _TM, _TM), :]
            onehot = (cols == eid_t).astype(jnp.float32)
            base = jnp.sum(onehot * go_f, axis=-1, keepdims=True)
            pos_ref[pl.ds(tt * _TM, _TM), :] = (
                base + r1_s[pl.ds(tt * _TM, _TM), :]).astype(jnp.int32)

        ns = nt + e
        counts = carry
        inv_tm = 1.0 / _TM
        first_tile = jnp.floor(go_f * inv_tm)
        go_next = go_f + counts
        last_tile = jnp.floor((jnp.maximum(go_next, 1.0) - 1.0) * inv_tm)
        span = jnp.where(counts > 0, last_tile - first_tile + 1.0, 0.0)
        tri_ei = (gri <= gci).astype(jnp.float32)
        cumv = jnp.dot(span, tri_ei,
                       preferred_element_type=jnp.float32)
        slot_start = cumv - span
        total = cumv[:, e - 1:e]
        slot_f = lax.broadcasted_iota(
            jnp.int32, (ns, 1), 0).astype(jnp.float32)
        gi = jnp.sum((cumv <= slot_f).astype(jnp.float32),
                     axis=-1, keepdims=True)
        gi_i = jnp.minimum(gi, float(e - 1)).astype(jnp.int32)
        cols_s = lax.broadcasted_iota(jnp.int32, (ns, e), 1)
        ohs = (cols_s == gi_i).astype(jnp.float32)
        ft_i = jnp.sum(ohs * first_tile, axis=-1, keepdims=True)
        ss_i = jnp.sum(ohs * slot_start, axis=-1, keepdims=True)
        lo_i = jnp.sum(ohs * go_f, axis=-1, keepdims=True)
        hi_i = jnp.sum(ohs * go_next, axis=-1, keepdims=True)
        tile_i = ft_i + (slot_f - ss_i)
        valid = slot_f < total
        tile_last = jnp.max(jnp.where(valid, tile_i, -1.0),
                            axis=0, keepdims=True)
        g_last = jnp.max(jnp.where(valid, gi, -1.0), axis=0, keepdims=True)
        t_meta = jnp.where(valid, tile_i, tile_last).astype(jnp.int32)
        gmeta_ref[...] = jnp.where(valid, gi_i, g_last.astype(jnp.int32))
        tmeta_ref[...] = t_meta
        lo_ref[...] = jnp.where(valid, lo_i, 0.0).astype(jnp.int32)
        hi_ref[...] = jnp.where(valid, hi_i, 0.0).astype(jnp.int32)
        prev = jnp.concatenate(
            [jnp.full((1, 1), -1, jnp.int32), t_meta[:-1, :]], axis=0)
        first_ref[...] = (valid & (t_meta != prev)).astype(jnp.int32)

    ns_ = nt + e
    return pl.pallas_call(
        body,
        out_shape=(jax.ShapeDtypeStruct((t, 128), jnp.float32),
                   jax.ShapeDtypeStruct((t, 1), jnp.int32),
                   jax.ShapeDtypeStruct((ns_, 1), jnp.int32),
                   jax.ShapeDtypeStruct((ns_, 1), jnp.int32),
                   jax.ShapeDtypeStruct((ns_, 1), jnp.int32),
                   jax.ShapeDtypeStruct((ns_, 1), jnp.int32),
                   jax.ShapeDtypeStruct((ns_, 1), jnp.int32)),
        scratch_shapes=[pltpu.VMEM((t, 1), jnp.int32),
                        pltpu.VMEM((t, 1), jnp.float32)],
    )(x, w_router)


def _sc_dispatch(x, pos3, gate16):
    t, d = x.shape
    rows_per_w = t // _SC_WORKERS
    n_ch = rows_per_w // _SC_CHUNK
    mesh = plsc.VectorSubcoreMesh(core_axis_name="c", subcore_axis_name="s")

    @functools.partial(
        pl.kernel, mesh=mesh,
        out_type=(jax.ShapeDtypeStruct((t, d), jnp.float32),
                  jax.ShapeDtypeStruct((t, 128), jnp.float32)),
        scratch_types=[
            pltpu.VMEM((n_ch, _SC_CHUNK), jnp.int32),
            pltpu.VMEM((_SC_CHUNK, d), jnp.float32),
            pltpu.VMEM((_SC_CHUNK, d), jnp.float32),
            pltpu.VMEM((_SC_CHUNK, 128), jnp.float32),
            pltpu.VMEM((_SC_CHUNK, 128), jnp.float32),
            pltpu.SemaphoreType.DMA,
        ],
    )
    def k(x_hbm, pos3_hbm, g16_hbm, xs_hbm, g16s_hbm,
          pos2_v, rows_a, rows_b, grows_a, grows_b, sem):
        wid = lax.axis_index("s") * _SC_CORES + lax.axis_index("c")
        base = wid * rows_per_w
        pltpu.sync_copy(pos3_hbm.at[wid], pos2_v)
        bufs = [(rows_a, grows_a), (rows_b, grows_b)]
        pend = [None, None]
        for c in range(n_ch):
            rv, gv = bufs[c % 2]
            if pend[c % 2] is not None:
                for cp in pend[c % 2]:
                    cp.wait()
            pltpu.sync_copy(x_hbm.at[pl.ds(base + c * _SC_CHUNK, _SC_CHUNK)],
                            rv)
            pltpu.sync_copy(g16_hbm.at[pl.ds(base + c * _SC_CHUNK, _SC_CHUNK)],
                            gv)
            pend[c % 2] = (
                pltpu.async_copy(rv, xs_hbm.at[pos2_v.at[c]], sem),
                pltpu.async_copy(gv, g16s_hbm.at[pos2_v.at[c]], sem))
        for p in pend:
            if p is not None:
                for cp in p:
                    cp.wait()

    return k(x, pos3, gate16)


def _sc_gather(x, idx):
    t, d = x.shape
    rows_per_w = t // _SC_WORKERS
    n_ch = rows_per_w // _SC_CHUNK
    mesh = plsc.VectorSubcoreMesh(core_axis_name="c", subcore_axis_name="s")

    @functools.partial(
        pl.kernel, mesh=mesh,
        out_type=jax.ShapeDtypeStruct((t, d), jnp.float32),
        scratch_types=[
            pltpu.VMEM((rows_per_w,), jnp.int32),
            pltpu.VMEM((_SC_CHUNK, d), jnp.float32),
            pltpu.VMEM((_SC_CHUNK, d), jnp.float32),
            pltpu.SemaphoreType.DMA,
        ],
    )
    def k(x_hbm, idx_hbm, out_hbm, idx_v, rows_a, rows_b, sem):
        wid = lax.axis_index("s") * _SC_CORES + lax.axis_index("c")
        base = wid * rows_per_w
        pltpu.sync_copy(idx_hbm.at[pl.ds(base, rows_per_w)], idx_v)
        bufs = [rows_a, rows_b]

        def gath(c):
            return pltpu.async_copy(
                x_hbm.at[idx_v.at[pl.ds(c * _SC_CHUNK, _SC_CHUNK)]],
                bufs[c % 2], sem)

        cps = [None] * n_ch
        for c in range(min(2, n_ch)):
            cps[c] = gath(c)
        for c in range(n_ch):
            cps[c].wait()
            pltpu.sync_copy(bufs[c % 2],
                            out_hbm.at[pl.ds(base + c * _SC_CHUNK,
                                             _SC_CHUNK)])
            if c + 2 < n_ch:
                cps[c + 2] = gath(c + 2)

    return k(x, idx)


def _grouped_mlp(xs, gate2, w1, b1r, w2, b2r, g_meta, t_meta, lo, hi, first):
    t, d = xs.shape
    e, _, f = w1.shape
    num_slots = g_meta.shape[0]

    def body(g_ref, t_ref, lo_ref, hi_ref, first_ref,
             xs_ref, w1_ref, b1_ref, w2_ref, b2_ref, gate_ref, out_ref):
        i = pl.program_id(0)
        row0 = t_ref[i] * _TM
        ridx = row0 + lax.broadcasted_iota(jnp.int32, (_TM, 1), 0)
        mask = (ridx >= lo_ref[i]) & (ridx < hi_ref[i])
        h = jnp.dot(xs_ref[...], w1_ref[0],
                    preferred_element_type=jnp.float32) + b1_ref[0]
        h = jax.nn.gelu(h)
        y = jnp.dot(h, w2_ref[0], preferred_element_type=jnp.float32) + b2_ref[0]
        scale = jnp.where(mask, gate_ref[:, :1], 0.0)
        contrib = y * scale

        @pl.when(first_ref[i] != 0)
        def _init():
            out_ref[...] = contrib

        @pl.when(first_ref[i] == 0)
        def _accum():
            out_ref[...] += contrib

    grid_spec = pltpu.PrefetchScalarGridSpec(
        num_scalar_prefetch=5,
        grid=(num_slots,),
        in_specs=[
            pl.BlockSpec((_TM, d), lambda i, g, tt, lo_, hi_, fr: (tt[i], 0)),
            pl.BlockSpec((1, d, f), lambda i, g, tt, lo_, hi_, fr: (g[i], 0, 0)),
            pl.BlockSpec((1, 1, f), lambda i, g, tt, lo_, hi_, fr: (g[i], 0, 0)),
            pl.BlockSpec((1, f, d), lambda i, g, tt, lo_, hi_, fr: (g[i], 0, 0)),
            pl.BlockSpec((1, 1, d), lambda i, g, tt, lo_, hi_, fr: (g[i], 0, 0)),
            pl.BlockSpec((_TM, 128), lambda i, g, tt, lo_, hi_, fr: (tt[i], 0)),
        ],
        out_specs=pl.BlockSpec((_TM, d), lambda i, g, tt, lo_, hi_, fr: (tt[i], 0)),
    )
    return pl.pallas_call(
        body,
        grid_spec=grid_spec,
        out_shape=jax.ShapeDtypeStruct((t, d), jnp.float32),
        compiler_params=pltpu.CompilerParams(
            dimension_semantics=("arbitrary",)),
    )(g_meta, t_meta, lo, hi, first, xs, w1, b1r, w2, b2r, gate2)


def kernel(hidden_states, W_router, W1, b1, W2, b2):
    s, b, d = hidden_states.shape
    e, _, f = W1.shape
    x = hidden_states.reshape(-1, d)
    t = x.shape[0]
    nt = t // _TM
    num_slots = nt + e

    gate16, pos, g_meta, t_meta, lo, hi, first = _router_pos(x, W_router)
    g_meta = g_meta.reshape(-1)
    t_meta = t_meta.reshape(-1)
    lo = lo.reshape(-1)
    hi = hi.reshape(-1)
    first = first.reshape(-1)

    pos_flat = pos.reshape(-1)
    rows_per_w = t // _SC_WORKERS
    n_ch = rows_per_w // _SC_CHUNK
    pos3 = pos_flat.reshape(_SC_WORKERS, n_ch, _SC_CHUNK)
    xs, g16s = _sc_dispatch(x, pos3, gate16)

    b1r = b1.reshape(e, 1, f)
    b2r = b2.reshape(e, 1, d)
    ys = _grouped_mlp(xs, g16s, W1, b1r, W2, b2r,
                      g_meta, t_meta, lo, hi, first)

    out = _sc_gather(ys, pos_flat)
    return out.reshape(s, b, d)


# speedup vs baseline: 1.2060x; 1.0293x over previous
import functools

import jax
import jax.numpy as jnp
from jax import lax
from jax.experimental import pallas as pl
from jax.experimental.pallas import tpu as pltpu
from jax.experimental.pallas import tpu_sc as plsc

_SC_CORES = 2
_SC_SUBCORES = 16
_SC_WORKERS = _SC_CORES * _SC_SUBCORES
_SC_CHUNK = 32

_TM = 256


def _router_pos(x, w_router):
    t, d = x.shape
    e = w_router.shape[1]
    nt = t // _TM

    def body(x_ref, wr_ref, gate16_ref, pos_ref, gmeta_ref, tmeta_ref,
             lo_ref, hi_ref, first_ref, eid_s, r1_s):
        logits = jnp.dot(x_ref[...], wr_ref[...],
                         preferred_element_type=jnp.float32)
        m = jnp.max(logits, axis=-1, keepdims=True)
        ssum = jnp.sum(jnp.exp(logits - m), axis=-1, keepdims=True)
        gate16_ref[...] = jnp.broadcast_to(1.0 / ssum, (t, 128))
        eid_s[...] = jnp.argmax(logits, axis=-1, keepdims=True).astype(
            jnp.int32)
        ri = lax.broadcasted_iota(jnp.int32, (_TM, _TM), 0)
        ci = lax.broadcasted_iota(jnp.int32, (_TM, _TM), 1)
        tri = (ci < ri).astype(jnp.float32)
        cols = lax.broadcasted_iota(jnp.int32, (_TM, e), 1)
        carry = jnp.zeros((1, e), jnp.float32)
        for tt in range(nt):
            eid_t = eid_s[pl.ds(tt * _TM, _TM), :]
            onehot = (cols == eid_t).astype(jnp.float32)
            cum = jnp.dot(tri, onehot, preferred_element_type=jnp.float32)
            r1_t = jnp.sum((cum + carry) * onehot, axis=-1, keepdims=True)
            r1_s[pl.ds(tt * _TM, _TM), :] = r1_t
            carry = carry + jnp.sum(onehot, axis=0, keepdims=True)
        gri = lax.broadcasted_iota(jnp.int32, (e, e), 0)
        gci = lax.broadcasted_iota(jnp.int32, (e, e), 1)
        tri_e = (gri < gci).astype(jnp.float32)
        go_f = jnp.dot(carry, tri_e,
                       preferred_element_type=jnp.float32)
        for tt in range(nt):
            eid_t = eid_s[pl.ds(tt * _TM, _TM), :]
            onehot = (cols == eid_t).astype(jnp.float32)
            base = jnp.sum(onehot * go_f, axis=-1, keepdims=True)
            pos_ref[pl.ds(tt * _TM, _TM), :] = (
                base + r1_s[pl.ds(tt * _TM, _TM), :]).astype(jnp.int32)

        ns = nt + e
        counts = carry
        inv_tm = 1.0 / _TM
        first_tile = jnp.floor(go_f * inv_tm)
        go_next = go_f + counts
        last_tile = jnp.floor((jnp.maximum(go_next, 1.0) - 1.0) * inv_tm)
        span = jnp.where(counts > 0, last_tile - first_tile + 1.0, 0.0)
        tri_ei = (gri <= gci).astype(jnp.float32)
        cumv = jnp.dot(span, tri_ei,
                       preferred_element_type=jnp.float32)
        slot_start = cumv - span
        total = cumv[:, e - 1:e]
        slot_f = lax.broadcasted_iota(
            jnp.int32, (ns, 1), 0).astype(jnp.float32)
        gi = jnp.sum((cumv <= slot_f).astype(jnp.float32),
                     axis=-1, keepdims=True)
        gi_i = jnp.minimum(gi, float(e - 1)).astype(jnp.int32)
        cols_s = lax.broadcasted_iota(jnp.int32, (ns, e), 1)
        ohs = (cols_s == gi_i).astype(jnp.float32)
        ft_i = jnp.sum(ohs * first_tile, axis=-1, keepdims=True)
        ss_i = jnp.sum(ohs * slot_start, axis=-1, keepdims=True)
        lo_i = jnp.sum(ohs * go_f, axis=-1, keepdims=True)
        hi_i = jnp.sum(ohs * go_next, axis=-1, keepdims=True)
        tile_i = ft_i + (slot_f - ss_i)
        valid = slot_f < total
        tile_last = jnp.max(jnp.where(valid, tile_i, -1.0),
                            axis=0, keepdims=True)
        g_last = jnp.max(jnp.where(valid, gi, -1.0), axis=0, keepdims=True)
        t_meta = jnp.where(valid, tile_i, tile_last).astype(jnp.int32)
        gmeta_ref[...] = jnp.where(valid, gi_i, g_last.astype(jnp.int32))
        tmeta_ref[...] = t_meta
        lo_ref[...] = jnp.where(valid, lo_i, 0.0).astype(jnp.int32)
        hi_ref[...] = jnp.where(valid, hi_i, 0.0).astype(jnp.int32)
        prev = jnp.concatenate(
            [jnp.full((1, 1), -1, jnp.int32), t_meta[:-1, :]], axis=0)
        first_ref[...] = (valid & (t_meta != prev)).astype(jnp.int32)

    ns_ = nt + e
    return pl.pallas_call(
        body,
        out_shape=(jax.ShapeDtypeStruct((t, 128), jnp.float32),
                   jax.ShapeDtypeStruct((t, 1), jnp.int32),
                   jax.ShapeDtypeStruct((ns_, 1), jnp.int32),
                   jax.ShapeDtypeStruct((ns_, 1), jnp.int32),
                   jax.ShapeDtypeStruct((ns_, 1), jnp.int32),
                   jax.ShapeDtypeStruct((ns_, 1), jnp.int32),
                   jax.ShapeDtypeStruct((ns_, 1), jnp.int32)),
        scratch_shapes=[pltpu.VMEM((t, 1), jnp.int32),
                        pltpu.VMEM((t, 1), jnp.float32)],
    )(x, w_router)


def _sc_dispatch(x, pos3, gate16):
    t, d = x.shape
    rows_per_w = t // _SC_WORKERS
    n_ch = rows_per_w // _SC_CHUNK
    mesh = plsc.VectorSubcoreMesh(core_axis_name="c", subcore_axis_name="s")

    @functools.partial(
        pl.kernel, mesh=mesh,
        out_type=(jax.ShapeDtypeStruct((t, d), jnp.float32),
                  jax.ShapeDtypeStruct((t, 128), jnp.float32)),
        scratch_types=[
            pltpu.VMEM((n_ch, _SC_CHUNK), jnp.int32),
            pltpu.VMEM((_SC_CHUNK, d), jnp.float32),
            pltpu.VMEM((_SC_CHUNK, d), jnp.float32),
            pltpu.VMEM((_SC_CHUNK, 128), jnp.float32),
            pltpu.VMEM((_SC_CHUNK, 128), jnp.float32),
            pltpu.SemaphoreType.DMA,
        ],
    )
    def k(x_hbm, pos3_hbm, g16_hbm, xs_hbm, g16s_hbm,
          pos2_v, rows_a, rows_b, grows_a, grows_b, sem):
        wid = lax.axis_index("s") * _SC_CORES + lax.axis_index("c")
        base = wid * rows_per_w
        pltpu.sync_copy(pos3_hbm.at[wid], pos2_v)
        bufs = [(rows_a, grows_a), (rows_b, grows_b)]
        pend = [None, None]
        for c in range(n_ch):
            rv, gv = bufs[c % 2]
            if pend[c % 2] is not None:
                for cp in pend[c % 2]:
                    cp.wait()
            pltpu.sync_copy(x_hbm.at[pl.ds(base + c * _SC_CHUNK, _SC_CHUNK)],
                            rv)
            pltpu.sync_copy(g16_hbm.at[pl.ds(base + c * _SC_CHUNK, _SC_CHUNK)],
                            gv)
            pend[c % 2] = (
                pltpu.async_copy(rv, xs_hbm.at[pos2_v.at[c]], sem),
                pltpu.async_copy(gv, g16s_hbm.at[pos2_v.at[c]], sem))
        for p in pend:
            if p is not None:
                for cp in p:
                    cp.wait()

    return k(x, pos3, gate16)


def _sc_gather(x, idx):
    t, d = x.shape
    rows_per_w = t // _SC_WORKERS
    n_ch = rows_per_w // _SC_CHUNK
    mesh = plsc.VectorSubcoreMesh(core_axis_name="c", subcore_axis_name="s")

    @functools.partial(
        pl.kernel, mesh=mesh,
        out_type=jax.ShapeDtypeStruct((t, d), jnp.float32),
        scratch_types=[
            pltpu.VMEM((rows_per_w,), jnp.int32),
            pltpu.VMEM((_SC_CHUNK, d), jnp.float32),
            pltpu.VMEM((_SC_CHUNK, d), jnp.float32),
            pltpu.SemaphoreType.DMA,
        ],
    )
    def k(x_hbm, idx_hbm, out_hbm, idx_v, rows_a, rows_b, sem):
        wid = lax.axis_index("s") * _SC_CORES + lax.axis_index("c")
        base = wid * rows_per_w
        pltpu.sync_copy(idx_hbm.at[pl.ds(base, rows_per_w)], idx_v)
        bufs = [rows_a, rows_b]

        def gath(c):
            return pltpu.async_copy(
                x_hbm.at[idx_v.at[pl.ds(c * _SC_CHUNK, _SC_CHUNK)]],
                bufs[c % 2], sem)

        cps = [None] * n_ch
        for c in range(min(2, n_ch)):
            cps[c] = gath(c)
        for c in range(n_ch):
            cps[c].wait()
            pltpu.sync_copy(bufs[c % 2],
                            out_hbm.at[pl.ds(base + c * _SC_CHUNK,
                                             _SC_CHUNK)])
            if c + 2 < n_ch:
                cps[c + 2] = gath(c + 2)

    return k(x, idx)


def _grouped_mlp(xs, gate2, w1, b1r, w2, b2r, g_meta, t_meta, lo, hi, first):
    t, d = xs.shape
    e, _, f = w1.shape
    num_slots = g_meta.shape[0]

    def body(g_ref, t_ref, lo_ref, hi_ref, first_ref,
             xs_ref, w1_ref, b1_ref, w2_ref, b2_ref, gate_ref, out_ref):
        i = pl.program_id(0)
        row0 = t_ref[i] * _TM
        ridx = row0 + lax.broadcasted_iota(jnp.int32, (_TM, 1), 0)
        mask = (ridx >= lo_ref[i]) & (ridx < hi_ref[i])
        h = jnp.dot(xs_ref[...], w1_ref[0],
                    preferred_element_type=jnp.float32) + b1_ref[0]
        h = jax.nn.gelu(h)
        y = jnp.dot(h, w2_ref[0], preferred_element_type=jnp.float32) + b2_ref[0]
        scale = jnp.where(mask, gate_ref[:, :1], 0.0)
        contrib = y * scale

        @pl.when(first_ref[i] != 0)
        def _init():
            out_ref[...] = contrib

        @pl.when(first_ref[i] == 0)
        def _accum():
            out_ref[...] += contrib

    grid_spec = pltpu.PrefetchScalarGridSpec(
        num_scalar_prefetch=5,
        grid=(num_slots,),
        in_specs=[
            pl.BlockSpec((_TM, d), lambda i, g, tt, lo_, hi_, fr: (tt[i], 0)),
            pl.BlockSpec((1, d, f), lambda i, g, tt, lo_, hi_, fr: (g[i], 0, 0)),
            pl.BlockSpec((1, 1, f), lambda i, g, tt, lo_, hi_, fr: (g[i], 0, 0)),
            pl.BlockSpec((1, f, d), lambda i, g, tt, lo_, hi_, fr: (g[i], 0, 0)),
            pl.BlockSpec((1, 1, d), lambda i, g, tt, lo_, hi_, fr: (g[i], 0, 0)),
            pl.BlockSpec((_TM, 128), lambda i, g, tt, lo_, hi_, fr: (tt[i], 0)),
        ],
        out_specs=pl.BlockSpec((_TM, d), lambda i, g, tt, lo_, hi_, fr: (tt[i], 0)),
    )
    return pl.pallas_call(
        body,
        grid_spec=grid_spec,
        out_shape=jax.ShapeDtypeStruct((t, d), jnp.float32),
        compiler_params=pltpu.CompilerParams(
            dimension_semantics=("arbitrary",)),
    )(g_meta, t_meta, lo, hi, first, xs, w1, b1r, w2, b2r, gate2)


def kernel(hidden_states, W_router, W1, b1, W2, b2):
    s, b, d = hidden_states.shape
    e, _, f = W1.shape
    x = hidden_states.reshape(-1, d)
    t = x.shape[0]
    nt = t // _TM
    num_slots = nt + e

    gate16, pos, g_meta, t_meta, lo, hi, first = _router_pos(x, W_router)
    g_meta = g_meta.reshape(-1)
    t_meta = t_meta.reshape(-1)
    lo = lo.reshape(-1)
    hi = hi.reshape(-1)
    first = first.reshape(-1)

    pos_flat = pos.reshape(-1)
    rows_per_w = t // _SC_WORKERS
    n_ch = rows_per_w // _SC_CHUNK
    pos3 = pos_flat.reshape(_SC_WORKERS, n_ch, _SC_CHUNK)
    xs, g16s = _sc_dispatch(x, pos3, gate16)

    b1r = b1.reshape(e, 1, f)
    b2r = b2.reshape(e, 1, d)
    ys = _grouped_mlp(xs, g16s, W1, b1r, W2, b2r,
                      g_meta, t_meta, lo, hi, first)

    out = _sc_gather(ys, pos_flat)
    return out.reshape(s, b, d)


# speedup vs baseline: 1.2210x; 1.0124x over previous
import functools

import jax
import jax.numpy as jnp
from jax import lax
from jax.experimental import pallas as pl
from jax.experimental.pallas import tpu as pltpu
from jax.experimental.pallas import tpu_sc as plsc

_SC_CORES = 2
_SC_SUBCORES = 16
_SC_WORKERS = _SC_CORES * _SC_SUBCORES
_SC_CHUNK = 32

_TM = 512


def _router_pos(x, w_router):
    t, d = x.shape
    e = w_router.shape[1]
    nt = t // _TM

    def body(x_ref, wr_ref, gate16_ref, pos_ref, gmeta_ref, tmeta_ref,
             lo_ref, hi_ref, first_ref, eid_s, r1_s):
        logits = jnp.dot(x_ref[...], wr_ref[...],
                         preferred_element_type=jnp.float32)
        m = jnp.max(logits, axis=-1, keepdims=True)
        ssum = jnp.sum(jnp.exp(logits - m), axis=-1, keepdims=True)
        gate16_ref[...] = jnp.broadcast_to(1.0 / ssum, (t, 128))
        eid_s[...] = jnp.argmax(logits, axis=-1, keepdims=True).astype(
            jnp.int32)
        ri = lax.broadcasted_iota(jnp.int32, (_TM, _TM), 0)
        ci = lax.broadcasted_iota(jnp.int32, (_TM, _TM), 1)
        tri = (ci < ri).astype(jnp.float32)
        cols = lax.broadcasted_iota(jnp.int32, (_TM, e), 1)
        carry = jnp.zeros((1, e), jnp.float32)
        for tt in range(nt):
            eid_t = eid_s[pl.ds(tt * _TM, _TM), :]
            onehot = (cols == eid_t).astype(jnp.float32)
            cum = jnp.dot(tri, onehot, preferred_element_type=jnp.float32)
            r1_t = jnp.sum((cum + carry) * onehot, axis=-1, keepdims=True)
            r1_s[pl.ds(tt * _TM, _TM), :] = r1_t
            carry = carry + jnp.sum(onehot, axis=0, keepdims=True)
        gri = lax.broadcasted_iota(jnp.int32, (e, e), 0)
        gci = lax.broadcasted_iota(jnp.int32, (e, e), 1)
        tri_e = (gri < gci).astype(jnp.float32)
        go_f = jnp.dot(carry, tri_e,
                       preferred_element_type=jnp.float32)
        for tt in range(nt):
            eid_t = eid_s[pl.ds(tt * _TM, _TM), :]
            onehot = (cols == eid_t).astype(jnp.float32)
            base = jnp.sum(onehot * go_f, axis=-1, keepdims=True)
            pos_ref[pl.ds(tt * _TM, _TM), :] = (
                base + r1_s[pl.ds(tt * _TM, _TM), :]).astype(jnp.int32)

        ns = nt + e
        counts = carry
        inv_tm = 1.0 / _TM
        first_tile = jnp.floor(go_f * inv_tm)
        go_next = go_f + counts
        last_tile = jnp.floor((jnp.maximum(go_next, 1.0) - 1.0) * inv_tm)
        span = jnp.where(counts > 0, last_tile - first_tile + 1.0, 0.0)
        tri_ei = (gri <= gci).astype(jnp.float32)
        cumv = jnp.dot(span, tri_ei,
                       preferred_element_type=jnp.float32)
        slot_start = cumv - span
        total = cumv[:, e - 1:e]
        slot_f = lax.broadcasted_iota(
            jnp.int32, (ns, 1), 0).astype(jnp.float32)
        gi = jnp.sum((cumv <= slot_f).astype(jnp.float32),
                     axis=-1, keepdims=True)
        gi_i = jnp.minimum(gi, float(e - 1)).astype(jnp.int32)
        cols_s = lax.broadcasted_iota(jnp.int32, (ns, e), 1)
        ohs = (cols_s == gi_i).astype(jnp.float32)
        ft_i = jnp.sum(ohs * first_tile, axis=-1, keepdims=True)
        ss_i = jnp.sum(ohs * slot_start, axis=-1, keepdims=True)
        lo_i = jnp.sum(ohs * go_f, axis=-1, keepdims=True)
        hi_i = jnp.sum(ohs * go_next, axis=-1, keepdims=True)
        tile_i = ft_i + (slot_f - ss_i)
        valid = slot_f < total
        tile_last = jnp.max(jnp.where(valid, tile_i, -1.0),
                            axis=0, keepdims=True)
        g_last = jnp.max(jnp.where(valid, gi, -1.0), axis=0, keepdims=True)
        t_meta = jnp.where(valid, tile_i, tile_last).astype(jnp.int32)
        gmeta_ref[...] = jnp.where(valid, gi_i, g_last.astype(jnp.int32))
        tmeta_ref[...] = t_meta
        lo_ref[...] = jnp.where(valid, lo_i, 0.0).astype(jnp.int32)
        hi_ref[...] = jnp.where(valid, hi_i, 0.0).astype(jnp.int32)
        prev = jnp.concatenate(
            [jnp.full((1, 1), -1, jnp.int32), t_meta[:-1, :]], axis=0)
        first_ref[...] = (valid & (t_meta != prev)).astype(jnp.int32)

    ns_ = nt + e
    return pl.pallas_call(
        body,
        out_shape=(jax.ShapeDtypeStruct((t, 128), jnp.float32),
                   jax.ShapeDtypeStruct((t, 1), jnp.int32),
                   jax.ShapeDtypeStruct((ns_, 1), jnp.int32),
                   jax.ShapeDtypeStruct((ns_, 1), jnp.int32),
                   jax.ShapeDtypeStruct((ns_, 1), jnp.int32),
                   jax.ShapeDtypeStruct((ns_, 1), jnp.int32),
                   jax.ShapeDtypeStruct((ns_, 1), jnp.int32)),
        scratch_shapes=[pltpu.VMEM((t, 1), jnp.int32),
                        pltpu.VMEM((t, 1), jnp.float32)],
    )(x, w_router)


def _sc_dispatch(x, pos3, gate16):
    t, d = x.shape
    rows_per_w = t // _SC_WORKERS
    n_ch = rows_per_w // _SC_CHUNK
    mesh = plsc.VectorSubcoreMesh(core_axis_name="c", subcore_axis_name="s")

    @functools.partial(
        pl.kernel, mesh=mesh,
        out_type=(jax.ShapeDtypeStruct((t, d), jnp.float32),
                  jax.ShapeDtypeStruct((t, 128), jnp.float32)),
        scratch_types=[
            pltpu.VMEM((n_ch, _SC_CHUNK), jnp.int32),
            pltpu.VMEM((_SC_CHUNK, d), jnp.float32),
            pltpu.VMEM((_SC_CHUNK, d), jnp.float32),
            pltpu.VMEM((_SC_CHUNK, 128), jnp.float32),
            pltpu.VMEM((_SC_CHUNK, 128), jnp.float32),
            pltpu.SemaphoreType.DMA,
        ],
    )
    def k(x_hbm, pos3_hbm, g16_hbm, xs_hbm, g16s_hbm,
          pos2_v, rows_a, rows_b, grows_a, grows_b, sem):
        wid = lax.axis_index("s") * _SC_CORES + lax.axis_index("c")
        base = wid * rows_per_w
        pltpu.sync_copy(pos3_hbm.at[wid], pos2_v)
        bufs = [(rows_a, grows_a), (rows_b, grows_b)]
        pend = [None, None]
        for c in range(n_ch):
            rv, gv = bufs[c % 2]
            if pend[c % 2] is not None:
                for cp in pend[c % 2]:
                    cp.wait()
            pltpu.sync_copy(x_hbm.at[pl.ds(base + c * _SC_CHUNK, _SC_CHUNK)],
                            rv)
            pltpu.sync_copy(g16_hbm.at[pl.ds(base + c * _SC_CHUNK, _SC_CHUNK)],
                            gv)
            pend[c % 2] = (
                pltpu.async_copy(rv, xs_hbm.at[pos2_v.at[c]], sem),
                pltpu.async_copy(gv, g16s_hbm.at[pos2_v.at[c]], sem))
        for p in pend:
            if p is not None:
                for cp in p:
                    cp.wait()

    return k(x, pos3, gate16)


def _sc_gather(x, idx):
    t, d = x.shape
    rows_per_w = t // _SC_WORKERS
    n_ch = rows_per_w // _SC_CHUNK
    mesh = plsc.VectorSubcoreMesh(core_axis_name="c", subcore_axis_name="s")

    @functools.partial(
        pl.kernel, mesh=mesh,
        out_type=jax.ShapeDtypeStruct((t, d), jnp.float32),
        scratch_types=[
            pltpu.VMEM((rows_per_w,), jnp.int32),
            pltpu.VMEM((_SC_CHUNK, d), jnp.float32),
            pltpu.VMEM((_SC_CHUNK, d), jnp.float32),
            pltpu.SemaphoreType.DMA,
        ],
    )
    def k(x_hbm, idx_hbm, out_hbm, idx_v, rows_a, rows_b, sem):
        wid = lax.axis_index("s") * _SC_CORES + lax.axis_index("c")
        base = wid * rows_per_w
        pltpu.sync_copy(idx_hbm.at[pl.ds(base, rows_per_w)], idx_v)
        bufs = [rows_a, rows_b]

        def gath(c):
            return pltpu.async_copy(
                x_hbm.at[idx_v.at[pl.ds(c * _SC_CHUNK, _SC_CHUNK)]],
                bufs[c % 2], sem)

        cps = [None] * n_ch
        for c in range(min(2, n_ch)):
            cps[c] = gath(c)
        for c in range(n_ch):
            cps[c].wait()
            pltpu.sync_copy(bufs[c % 2],
                            out_hbm.at[pl.ds(base + c * _SC_CHUNK,
                                             _SC_CHUNK)])
            if c + 2 < n_ch:
                cps[c + 2] = gath(c + 2)

    return k(x, idx)


def _grouped_mlp(xs, gate2, w1, b1r, w2, b2r, g_meta, t_meta, lo, hi, first):
    t, d = xs.shape
    e, _, f = w1.shape
    num_slots = g_meta.shape[0]

    def body(g_ref, t_ref, lo_ref, hi_ref, first_ref,
             xs_ref, w1_ref, b1_ref, w2_ref, b2_ref, gate_ref, out_ref):
        i = pl.program_id(0)
        row0 = t_ref[i] * _TM
        ridx = row0 + lax.broadcasted_iota(jnp.int32, (_TM, 1), 0)
        mask = (ridx >= lo_ref[i]) & (ridx < hi_ref[i])
        h = jnp.dot(xs_ref[...], w1_ref[0],
                    preferred_element_type=jnp.float32) + b1_ref[0]
        h = jax.nn.gelu(h)
        y = jnp.dot(h, w2_ref[0], preferred_element_type=jnp.float32) + b2_ref[0]
        scale = jnp.where(mask, gate_ref[:, :1], 0.0)
        contrib = y * scale

        @pl.when(first_ref[i] != 0)
        def _init():
            out_ref[...] = contrib

        @pl.when(first_ref[i] == 0)
        def _accum():
            out_ref[...] += contrib

    grid_spec = pltpu.PrefetchScalarGridSpec(
        num_scalar_prefetch=5,
        grid=(num_slots,),
        in_specs=[
            pl.BlockSpec((_TM, d), lambda i, g, tt, lo_, hi_, fr: (tt[i], 0)),
            pl.BlockSpec((1, d, f), lambda i, g, tt, lo_, hi_, fr: (g[i], 0, 0)),
            pl.BlockSpec((1, 1, f), lambda i, g, tt, lo_, hi_, fr: (g[i], 0, 0)),
            pl.BlockSpec((1, f, d), lambda i, g, tt, lo_, hi_, fr: (g[i], 0, 0)),
            pl.BlockSpec((1, 1, d), lambda i, g, tt, lo_, hi_, fr: (g[i], 0, 0)),
            pl.BlockSpec((_TM, 128), lambda i, g, tt, lo_, hi_, fr: (tt[i], 0)),
        ],
        out_specs=pl.BlockSpec((_TM, d), lambda i, g, tt, lo_, hi_, fr: (tt[i], 0)),
    )
    return pl.pallas_call(
        body,
        grid_spec=grid_spec,
        out_shape=jax.ShapeDtypeStruct((t, d), jnp.float32),
        compiler_params=pltpu.CompilerParams(
            dimension_semantics=("arbitrary",)),
    )(g_meta, t_meta, lo, hi, first, xs, w1, b1r, w2, b2r, gate2)


def kernel(hidden_states, W_router, W1, b1, W2, b2):
    s, b, d = hidden_states.shape
    e, _, f = W1.shape
    x = hidden_states.reshape(-1, d)
    t = x.shape[0]
    nt = t // _TM
    num_slots = nt + e

    gate16, pos, g_meta, t_meta, lo, hi, first = _router_pos(x, W_router)
    g_meta = g_meta.reshape(-1)
    t_meta = t_meta.reshape(-1)
    lo = lo.reshape(-1)
    hi = hi.reshape(-1)
    first = first.reshape(-1)

    pos_flat = pos.reshape(-1)
    rows_per_w = t // _SC_WORKERS
    n_ch = rows_per_w // _SC_CHUNK
    pos3 = pos_flat.reshape(_SC_WORKERS, n_ch, _SC_CHUNK)
    xs, g16s = _sc_dispatch(x, pos3, gate16)

    b1r = b1.reshape(e, 1, f)
    b2r = b2.reshape(e, 1, d)
    ys = _grouped_mlp(xs, g16s, W1, b1r, W2, b2r,
                      g_meta, t_meta, lo, hi, first)

    out = _sc_gather(ys, pos_flat)
    return out.reshape(s, b, d)
